# baseline (speedup 1.0000x reference)
import functools

import jax
import jax.numpy as jnp
from jax import lax
from jax.experimental import pallas as pl
from jax.experimental.pallas import tpu as pltpu
from jax.experimental.pallas import tpu_sc as plsc

_N = 10000
_E = 320000
_D = 128
_H = 256
_HH = 128
_L = 3
_P = 100000
_HID = 128

_NC = 2
_NS = 16
_NW = _NC * _NS

_ECH = 128
_ECN = 160
_SCL = 32
_SCN = _ECN // _SCL
_EPT = _ECH * _ECN
_EPAD = _NS * _EPT

_APAD = 10240
_RPT = _APAD // _NS

_PCH = 128
_PCN = 32
_PPT = _PCH * _PCN
_PPAD = _NW * _PPT

_mesh = plsc.VectorSubcoreMesh(
    core_axis_name="c", subcore_axis_name="s",
    num_cores=_NC, num_subcores=_NS)


@functools.partial(
    pl.kernel,
    out_type=jax.ShapeDtypeStruct((_APAD,), jnp.float32),
    mesh=_mesh,
    scratch_types=[
        pltpu.VMEM((_SCL, _ECH), jnp.int32),
        pltpu.VMEM((_ECH,), jnp.float32),
        pltpu.VMEM((_RPT,), jnp.float32),
        pltpu.VMEM_SHARED((_APAD,), jnp.float32),
    ],
)
def _deg_kernel(dst_hbm, invdeg_hbm, dst_v, ones_v, work_v, dacc):
    c = lax.axis_index("c")
    t = lax.axis_index("s")

    @pl.when(c == 0)
    def _():
        def fz(i, carry):
            work_v[pl.ds(i * 16, 16)] = jnp.zeros((16,), jnp.float32)
            return carry
        lax.fori_loop(0, _RPT // 16, fz, 0)

        def fo(i, carry):
            ones_v[pl.ds(i * 16, 16)] = jnp.full((16,), 1.0, jnp.float32)
            return carry
        lax.fori_loop(0, _ECH // 16, fo, 0)

        pltpu.sync_copy(work_v, dacc.at[pl.ds(t * _RPT, _RPT)])
        plsc.subcore_barrier()

        def superchunk(s, carry):
            pltpu.sync_copy(
                dst_hbm.at[pl.ds(t * _ECN + s * _SCL, _SCL)], dst_v)

            def body(j, carry2):
                pltpu.sync_copy(ones_v, dacc.at[dst_v.at[j]], add=True)
                return carry2
            lax.fori_loop(0, _SCL, body, 0)
            return carry
        lax.fori_loop(0, _SCN, superchunk, 0)
        plsc.subcore_barrier()

        pltpu.sync_copy(dacc.at[pl.ds(t * _RPT, _RPT)], work_v)

        def finv(i, carry):
            v = work_v[pl.ds(i * 16, 16)]
            work_v[pl.ds(i * 16, 16)] = 1.0 / jnp.maximum(v, 1.0)
            return carry
        lax.fori_loop(0, _RPT // 16, finv, 0)
        pltpu.sync_copy(work_v, invdeg_hbm.at[pl.ds(t * _RPT, _RPT)])


@functools.partial(
    pl.kernel,
    out_type=jax.ShapeDtypeStruct((2 * _APAD, _HH), jnp.float32),
    mesh=_mesh,
    scratch_types=[
        pltpu.VMEM((_SCL, _ECH), jnp.int32),
        pltpu.VMEM((_SCL, _ECH), jnp.int32),
        pltpu.VMEM((_ECH, _HH), jnp.float32),
        pltpu.VMEM((_ECH, _HH), jnp.float32),
        pltpu.VMEM_SHARED((_APAD, _HH), jnp.float32),
        pltpu.SemaphoreType.DMA,
        pltpu.SemaphoreType.DMA,
    ],
)
def _agg_kernel(h2_hbm, src_hbm, dst_hbm, out_hbm,
                src_v, dst_v, rows0, rows1, acc, sem0, sem1):
    c = lax.axis_index("c")
    t = lax.axis_index("s")

    def fz(i, carry):
        r = i // 8
        col = (i - r * 8) * 16
        rows0[r, pl.ds(col, 16)] = jnp.zeros((16,), jnp.float32)
        return carry
    lax.fori_loop(0, _ECH * 8, fz, 0)
    for k in range(_RPT // _ECH):
        pltpu.sync_copy(rows0, acc.at[pl.ds(t * _RPT + k * _ECH, _ECH)])
    plsc.subcore_barrier()

    off = c * _N
    base = t * _ECN

    def superchunk(s, carry):
        pltpu.sync_copy(src_hbm.at[pl.ds(base + s * _SCL, _SCL)], src_v)
        pltpu.sync_copy(dst_hbm.at[pl.ds(base + s * _SCL, _SCL)], dst_v)

        def fs(i, carry2):
            r = i // 8
            col = (i - r * 8) * 16
            src_v[r, pl.ds(col, 16)] = src_v[r, pl.ds(col, 16)] + off
            return carry2
        lax.fori_loop(0, _SCL * 8, fs, 0)

        def g_start(j, buf, sem):
            pltpu.make_async_copy(h2_hbm.at[src_v.at[j]], buf, sem).start()

        def g_wait(j, buf, sem):
            pltpu.make_async_copy(h2_hbm.at[src_v.at[j]], buf, sem).wait()

        g_start(0, rows0, sem0)
        g_start(1, rows1, sem1)

        def body(k, carry2):
            j0 = 2 * k
            g_wait(j0, rows0, sem0)

            @pl.when(j0 + 2 < _SCL)
            def _():
                g_start(j0 + 2, rows0, sem0)
            pltpu.sync_copy(rows0, acc.at[dst_v.at[j0]], add=True)

            g_wait(j0 + 1, rows1, sem1)

            @pl.when(j0 + 3 < _SCL)
            def _():
                g_start(j0 + 3, rows1, sem1)
            pltpu.sync_copy(rows1, acc.at[dst_v.at[j0 + 1]], add=True)
            return carry2
        lax.fori_loop(0, _SCL // 2, body, 0)
        return carry
    lax.fori_loop(0, _SCN, superchunk, 0)
    plsc.subcore_barrier()

    pltpu.sync_copy(acc.at[pl.ds(t * _RPT, _RPT)],
                    out_hbm.at[pl.ds(c * _APAD + t * _RPT, _RPT)])


@functools.partial(
    pl.kernel,
    out_type=(jax.ShapeDtypeStruct((_PPAD, _H), jnp.float32),
              jax.ShapeDtypeStruct((_PPAD, _H), jnp.float32)),
    mesh=_mesh,
    scratch_types=[
        pltpu.VMEM((_PCN, _PCH), jnp.int32),
        pltpu.VMEM((_PCN, _PCH), jnp.int32),
        pltpu.VMEM((_PCH, _H), jnp.float32),
        pltpu.VMEM((_PCH, _H), jnp.float32),
        pltpu.SemaphoreType.DMA,
        pltpu.SemaphoreType.DMA,
    ],
)
def _pair_kernel(hn_hbm, i_hbm, j_hbm, hi_hbm, hj_hbm,
                 iv, jv, bi, bj, si, sj):
    c = lax.axis_index("c")
    s = lax.axis_index("s")
    w = c * _NS + s
    base = w * _PPT

    pltpu.sync_copy(i_hbm.at[pl.ds(w * _PCN, _PCN)], iv)
    pltpu.sync_copy(j_hbm.at[pl.ds(w * _PCN, _PCN)], jv)

    def gi_start(j):
        pltpu.make_async_copy(hn_hbm.at[iv.at[j]], bi, si).start()

    def gi_wait(j):
        pltpu.make_async_copy(hn_hbm.at[iv.at[j]], bi, si).wait()

    def gj_start(j):
        pltpu.make_async_copy(hn_hbm.at[jv.at[j]], bj, sj).start()

    def gj_wait(j):
        pltpu.make_async_copy(hn_hbm.at[jv.at[j]], bj, sj).wait()

    gi_start(0)

    def body(k, carry):
        gi_wait(k)
        gj_start(k)
        pltpu.sync_copy(bi, hi_hbm.at[pl.ds(base + k * _PCH, _PCH)])
        gj_wait(k)

        @pl.when(k + 1 < _PCN)
        def _():
            gi_start(k + 1)
        pltpu.sync_copy(bj, hj_hbm.at[pl.ds(base + k * _PCH, _PCH)])
        return carry
    lax.fori_loop(0, _PCN, body, 0)


_RB = 1000
_MB = 1024


def _inp_body(x_ref, w_ref, b_ref, o_ref):
    o_ref[0] = jnp.maximum(
        jnp.dot(x_ref[...], w_ref[...], preferred_element_type=jnp.float32)
        + b_ref[...], 0.0)


def _tc_input(X, W, b):
    return pl.pallas_call(
        _inp_body,
        grid=(2, 10),
        in_specs=[
            pl.BlockSpec((_RB, _D), lambda h, i: (i, 0)),
            pl.BlockSpec((_D, _HH), lambda h, i: (0, h)),
            pl.BlockSpec((1, _HH), lambda h, i: (0, h)),
        ],
        out_specs=pl.BlockSpec((1, _RB, _HH), lambda h, i: (h, i, 0)),
        out_shape=jax.ShapeDtypeStruct((2, _N, _HH), jnp.float32),
    )(X, W, b)


def _layer_math(h2_ref, agg_ref, inv_ref, ws_ref, wn_ref, b_ref, g_ref, lb_ref):
    h = jnp.concatenate([h2_ref[0], h2_ref[1]], axis=1)
    inv = inv_ref[...]
    m = jnp.concatenate([agg_ref[0] * inv, agg_ref[1] * inv], axis=1)
    out = (jnp.dot(h, ws_ref[...], preferred_element_type=jnp.float32)
           + jnp.dot(m, wn_ref[...], preferred_element_type=jnp.float32)
           + b_ref[...])
    out = jnp.maximum(out, 0.0)
    mu = jnp.mean(out, axis=1, keepdims=True)
    d = out - mu
    var = jnp.mean(d * d, axis=1, keepdims=True)
    return d * lax.rsqrt(var + 1e-5) * g_ref[...] + lb_ref[...]


def _layer_body(h2_ref, agg_ref, inv_ref, ws_ref, wn_ref, b_ref, g_ref,
                lb_ref, o_ref):
    hn = _layer_math(h2_ref, agg_ref, inv_ref, ws_ref, wn_ref, b_ref, g_ref,
                     lb_ref)
    o_ref[0] = hn[:, :_HH]
    o_ref[1] = hn[:, _HH:]


def _final_body(h2_ref, agg_ref, inv_ref, ws_ref, wn_ref, b_ref, g_ref,
                lb_ref, o_ref):
    hn = _layer_math(h2_ref, agg_ref, inv_ref, ws_ref, wn_ref, b_ref, g_ref,
                     lb_ref)
    nrm = jnp.sqrt(jnp.sum(hn * hn, axis=1, keepdims=True))
    o_ref[...] = hn / jnp.maximum(nrm, 1e-12)


def _tc_layer(h2, agg2, inv2d, ws, wn, b, g, lb, final):
    in_specs = [
        pl.BlockSpec((2, _RB, _HH), lambda i: (0, i, 0)),
        pl.BlockSpec((2, _RB, _HH), lambda i: (0, i, 0)),
        pl.BlockSpec((_RB, 1), lambda i: (i, 0)),
        pl.BlockSpec((_H, _H), lambda i: (0, 0)),
        pl.BlockSpec((_H, _H), lambda i: (0, 0)),
        pl.BlockSpec((1, _H), lambda i: (0, 0)),
        pl.BlockSpec((1, _H), lambda i: (0, 0)),
        pl.BlockSpec((1, _H), lambda i: (0, 0)),
    ]
    if final:
        out_specs = pl.BlockSpec((_RB, _H), lambda i: (i, 0))
        out_shape = jax.ShapeDtypeStruct((_N, _H), jnp.float32)
        body = _final_body
    else:
        out_specs = pl.BlockSpec((2, _RB, _HH), lambda i: (0, i, 0))
        out_shape = jax.ShapeDtypeStruct((2, _N, _HH), jnp.float32)
        body = _layer_body
    return pl.pallas_call(
        body, grid=(10,), in_specs=in_specs,
        out_specs=out_specs, out_shape=out_shape,
    )(h2, agg2, inv2d, ws, wn, b, g, lb)


def _mlp_body(hi_ref, hj_ref, w1_ref, b1_ref, w2_ref, b2_ref, w3_ref, b3_ref,
              o_ref):
    hi = hi_ref[...]
    hj = hj_ref[...]
    feat = jnp.concatenate([jnp.abs(hi - hj), hi * hj], axis=1)
    z = jnp.maximum(
        jnp.dot(feat, w1_ref[...], preferred_element_type=jnp.float32)
        + b1_ref[...], 0.0)
    z = jnp.maximum(
        jnp.dot(z, w2_ref[...], preferred_element_type=jnp.float32)
        + b2_ref[...], 0.0)
    o_ref[...] = (jnp.sum(z * w3_ref[...], axis=1, keepdims=True)
                  + b3_ref[...])


def _tc_mlp(HI, HJ, W1, b1, W2, b2, w3row, b3):
    return pl.pallas_call(
        _mlp_body,
        grid=(_PPAD // _MB,),
        in_specs=[
            pl.BlockSpec((_MB, _H), lambda i: (i, 0)),
            pl.BlockSpec((_MB, _H), lambda i: (i, 0)),
            pl.BlockSpec((2 * _H, _HID), lambda i: (0, 0)),
            pl.BlockSpec((1, _HID), lambda i: (0, 0)),
            pl.BlockSpec((_HID, _HID), lambda i: (0, 0)),
            pl.BlockSpec((1, _HID), lambda i: (0, 0)),
            pl.BlockSpec((1, _HID), lambda i: (0, 0)),
            pl.BlockSpec((1, 1), lambda i: (0, 0)),
        ],
        out_specs=pl.BlockSpec((_MB, 1), lambda i: (i, 0)),
        out_shape=jax.ShapeDtypeStruct((_PPAD, 1), jnp.float32),
    )(HI, HJ, W1, b1, W2, b2, w3row, b3)


def kernel(X, edge_index, i_idx, j_idx, W_inp, b_inp, Ws_self, bs_self,
           Ws_nei, bs_nei, ln_g, ln_b, W1, b1, W2, b2, W3, b3):
    src = edge_index[0]
    dst = edge_index[1]
    epad = _EPAD - _E
    src_slab = jnp.concatenate(
        [src, jnp.zeros((epad,), jnp.int32)]).reshape(_NS * _ECN, _ECH)
    dst_slab = jnp.concatenate(
        [dst, jnp.full((epad,), _N, jnp.int32)]).reshape(_NS * _ECN, _ECH)
    ppad = _PPAD - _P
    i_slab = jnp.concatenate(
        [i_idx, jnp.zeros((ppad,), jnp.int32)]).reshape(_NW * _PCN, _PCH)
    j_slab = jnp.concatenate(
        [j_idx, jnp.zeros((ppad,), jnp.int32)]).reshape(_NW * _PCN, _PCH)

    h2 = _tc_input(X, W_inp, b_inp.reshape(1, _H))
    invdeg = _deg_kernel(dst_slab)
    inv2d = invdeg[:_N].reshape(_N, 1)

    for l in range(_L):
        agg_flat = _agg_kernel(h2.reshape(2 * _N, _HH), src_slab, dst_slab)
        agg2 = agg_flat.reshape(2, _APAD, _HH)
        bsum = (bs_self[l] + bs_nei[l]).reshape(1, _H)
        if l < _L - 1:
            h2 = _tc_layer(h2, agg2, inv2d, Ws_self[l], Ws_nei[l], bsum,
                           ln_g[l].reshape(1, _H), ln_b[l].reshape(1, _H),
                           final=False)
        else:
            Hn = _tc_layer(h2, agg2, inv2d, Ws_self[l], Ws_nei[l], bsum,
                           ln_g[l].reshape(1, _H), ln_b[l].reshape(1, _H),
                           final=True)

    HI, HJ = _pair_kernel(Hn, i_slab, j_slab)
    out2d = _tc_mlp(HI, HJ, W1, b1.reshape(1, _HID), W2, b2.reshape(1, _HID),
                    W3.reshape(1, _HID), b3.reshape(1, 1))
    logits = out2d[:_P, 0]
    return (Hn, logits)


# speedup vs baseline: 1.0149x; 1.0149x over previous
import functools

import jax
import jax.numpy as jnp
from jax import lax
from jax.experimental import pallas as pl
from jax.experimental.pallas import tpu as pltpu
from jax.experimental.pallas import tpu_sc as plsc

_N = 10000
_E = 320000
_D = 128
_H = 256
_HH = 128
_L = 3
_P = 100000
_HID = 128

_NC = 2
_NS = 16
_NW = _NC * _NS

_ECH = 128
_ECN = 160
_SCL = 32
_SCN = _ECN // _SCL
_EPT = _ECH * _ECN
_EPAD = _NS * _EPT

_APAD = 10240
_RPT = _APAD // _NS

_PCH = 128
_PCN = 32
_PPT = _PCH * _PCN
_PPAD = _NW * _PPT

_mesh = plsc.VectorSubcoreMesh(
    core_axis_name="c", subcore_axis_name="s",
    num_cores=_NC, num_subcores=_NS)


@functools.partial(
    pl.kernel,
    out_type=jax.ShapeDtypeStruct((_APAD,), jnp.float32),
    mesh=_mesh,
    scratch_types=[
        pltpu.VMEM((_SCL, _ECH), jnp.int32),
        pltpu.VMEM((_ECH,), jnp.float32),
        pltpu.VMEM((_RPT,), jnp.float32),
        pltpu.VMEM_SHARED((_APAD,), jnp.float32),
    ],
)
def _deg_kernel(dst_hbm, invdeg_hbm, dst_v, ones_v, work_v, dacc):
    c = lax.axis_index("c")
    t = lax.axis_index("s")

    @pl.when(c == 0)
    def _():
        def fz(i, carry):
            work_v[pl.ds(i * 16, 16)] = jnp.zeros((16,), jnp.float32)
            return carry
        lax.fori_loop(0, _RPT // 16, fz, 0)

        def fo(i, carry):
            ones_v[pl.ds(i * 16, 16)] = jnp.full((16,), 1.0, jnp.float32)
            return carry
        lax.fori_loop(0, _ECH // 16, fo, 0)

        pltpu.sync_copy(work_v, dacc.at[pl.ds(t * _RPT, _RPT)])
        plsc.subcore_barrier()

        def superchunk(s, carry):
            pltpu.sync_copy(
                dst_hbm.at[pl.ds(t * _ECN + s * _SCL, _SCL)], dst_v)

            def body(j, carry2):
                pltpu.sync_copy(ones_v, dacc.at[dst_v.at[j]], add=True)
                return carry2
            lax.fori_loop(0, _SCL, body, 0)
            return carry
        lax.fori_loop(0, _SCN, superchunk, 0)
        plsc.subcore_barrier()

        pltpu.sync_copy(dacc.at[pl.ds(t * _RPT, _RPT)], work_v)

        def finv(i, carry):
            v = work_v[pl.ds(i * 16, 16)]
            work_v[pl.ds(i * 16, 16)] = 1.0 / jnp.maximum(v, 1.0)
            return carry
        lax.fori_loop(0, _RPT // 16, finv, 0)
        pltpu.sync_copy(work_v, invdeg_hbm.at[pl.ds(t * _RPT, _RPT)])


@functools.partial(
    pl.kernel,
    out_type=jax.ShapeDtypeStruct((2 * _APAD, _HH), jnp.float32),
    mesh=_mesh,
    scratch_types=[
        pltpu.VMEM((_SCL, _ECH), jnp.int32),
        pltpu.VMEM((_SCL, _ECH), jnp.int32),
        pltpu.VMEM((_ECH, _HH), jnp.float32),
        pltpu.VMEM((_ECH, _HH), jnp.float32),
        pltpu.VMEM_SHARED((_APAD, _HH), jnp.float32),
        pltpu.SemaphoreType.DMA,
        pltpu.SemaphoreType.DMA,
    ],
)
def _agg_kernel(h2_hbm, src_hbm, dst_hbm, out_hbm,
                src_v, dst_v, rows0, rows1, acc, sem0, sem1):
    c = lax.axis_index("c")
    t = lax.axis_index("s")

    def fz(i, carry):
        r = i // 8
        col = (i - r * 8) * 16
        rows0[r, pl.ds(col, 16)] = jnp.zeros((16,), jnp.float32)
        return carry
    lax.fori_loop(0, _ECH * 8, fz, 0)
    for k in range(_RPT // _ECH):
        pltpu.sync_copy(rows0, acc.at[pl.ds(t * _RPT + k * _ECH, _ECH)])
    plsc.subcore_barrier()

    off = c * _N
    base = t * _ECN

    def superchunk(s, carry):
        pltpu.sync_copy(src_hbm.at[pl.ds(base + s * _SCL, _SCL)], src_v)
        pltpu.sync_copy(dst_hbm.at[pl.ds(base + s * _SCL, _SCL)], dst_v)

        def fs(i, carry2):
            r = i // 8
            col = (i - r * 8) * 16
            src_v[r, pl.ds(col, 16)] = src_v[r, pl.ds(col, 16)] + off
            return carry2
        lax.fori_loop(0, _SCL * 8, fs, 0)

        def g_start(j, buf, sem):
            pltpu.make_async_copy(h2_hbm.at[src_v.at[j]], buf, sem).start()

        def g_wait(j, buf, sem):
            pltpu.make_async_copy(h2_hbm.at[src_v.at[j]], buf, sem).wait()

        g_start(0, rows0, sem0)
        g_start(1, rows1, sem1)

        def body(k, carry2):
            j0 = 2 * k
            g_wait(j0, rows0, sem0)

            @pl.when(j0 + 2 < _SCL)
            def _():
                g_start(j0 + 2, rows0, sem0)
            pltpu.sync_copy(rows0, acc.at[dst_v.at[j0]], add=True)

            g_wait(j0 + 1, rows1, sem1)

            @pl.when(j0 + 3 < _SCL)
            def _():
                g_start(j0 + 3, rows1, sem1)
            pltpu.sync_copy(rows1, acc.at[dst_v.at[j0 + 1]], add=True)
            return carry2
        lax.fori_loop(0, _SCL // 2, body, 0)
        return carry
    lax.fori_loop(0, _SCN, superchunk, 0)
    plsc.subcore_barrier()

    pltpu.sync_copy(acc.at[pl.ds(t * _RPT, _RPT)],
                    out_hbm.at[pl.ds(c * _APAD + t * _RPT, _RPT)])


@functools.partial(
    pl.kernel,
    out_type=(jax.ShapeDtypeStruct((_PPAD, _H), jnp.float32),
              jax.ShapeDtypeStruct((_PPAD, _H), jnp.float32)),
    mesh=_mesh,
    scratch_types=[
        pltpu.VMEM((_PCN, _PCH), jnp.int32),
        pltpu.VMEM((_PCN, _PCH), jnp.int32),
        pltpu.VMEM((_PCH, _H), jnp.float32),
        pltpu.VMEM((_PCH, _H), jnp.float32),
        pltpu.SemaphoreType.DMA,
        pltpu.SemaphoreType.DMA,
    ],
)
def _pair_kernel(hn_hbm, i_hbm, j_hbm, hi_hbm, hj_hbm,
                 iv, jv, bi, bj, si, sj):
    c = lax.axis_index("c")
    s = lax.axis_index("s")
    w = c * _NS + s
    base = w * _PPT

    pltpu.sync_copy(i_hbm.at[pl.ds(w * _PCN, _PCN)], iv)
    pltpu.sync_copy(j_hbm.at[pl.ds(w * _PCN, _PCN)], jv)

    def gi_start(j):
        pltpu.make_async_copy(hn_hbm.at[iv.at[j]], bi, si).start()

    def gi_wait(j):
        pltpu.make_async_copy(hn_hbm.at[iv.at[j]], bi, si).wait()

    def gj_start(j):
        pltpu.make_async_copy(hn_hbm.at[jv.at[j]], bj, sj).start()

    def gj_wait(j):
        pltpu.make_async_copy(hn_hbm.at[jv.at[j]], bj, sj).wait()

    gi_start(0)
    gj_start(0)

    def body(k, carry):
        gi_wait(k)
        pltpu.sync_copy(bi, hi_hbm.at[pl.ds(base + k * _PCH, _PCH)])

        @pl.when(k + 1 < _PCN)
        def _():
            gi_start(k + 1)
        gj_wait(k)
        pltpu.sync_copy(bj, hj_hbm.at[pl.ds(base + k * _PCH, _PCH)])

        @pl.when(k + 1 < _PCN)
        def _():
            gj_start(k + 1)
        return carry
    lax.fori_loop(0, _PCN, body, 0)


_RB = 1000
_MB = 1024


def _inp_body(x_ref, w_ref, b_ref, o_ref):
    o_ref[0] = jnp.maximum(
        jnp.dot(x_ref[...], w_ref[...], preferred_element_type=jnp.float32)
        + b_ref[...], 0.0)


def _tc_input(X, W, b):
    return pl.pallas_call(
        _inp_body,
        grid=(2, 10),
        in_specs=[
            pl.BlockSpec((_RB, _D), lambda h, i: (i, 0)),
            pl.BlockSpec((_D, _HH), lambda h, i: (0, h)),
            pl.BlockSpec((1, _HH), lambda h, i: (0, h)),
        ],
        out_specs=pl.BlockSpec((1, _RB, _HH), lambda h, i: (h, i, 0)),
        out_shape=jax.ShapeDtypeStruct((2, _N, _HH), jnp.float32),
    )(X, W, b)


def _layer_math(h2_ref, agg_ref, inv_ref, ws_ref, wn_ref, b_ref, g_ref, lb_ref):
    h = jnp.concatenate([h2_ref[0], h2_ref[1]], axis=1)
    inv = inv_ref[...]
    m = jnp.concatenate([agg_ref[0] * inv, agg_ref[1] * inv], axis=1)
    out = (jnp.dot(h, ws_ref[...], preferred_element_type=jnp.float32)
           + jnp.dot(m, wn_ref[...], preferred_element_type=jnp.float32)
           + b_ref[...])
    out = jnp.maximum(out, 0.0)
    mu = jnp.mean(out, axis=1, keepdims=True)
    d = out - mu
    var = jnp.mean(d * d, axis=1, keepdims=True)
    return d * lax.rsqrt(var + 1e-5) * g_ref[...] + lb_ref[...]


def _layer_body(h2_ref, agg_ref, inv_ref, ws_ref, wn_ref, b_ref, g_ref,
                lb_ref, o_ref):
    hn = _layer_math(h2_ref, agg_ref, inv_ref, ws_ref, wn_ref, b_ref, g_ref,
                     lb_ref)
    o_ref[0] = hn[:, :_HH]
    o_ref[1] = hn[:, _HH:]


def _final_body(h2_ref, agg_ref, inv_ref, ws_ref, wn_ref, b_ref, g_ref,
                lb_ref, o_ref):
    hn = _layer_math(h2_ref, agg_ref, inv_ref, ws_ref, wn_ref, b_ref, g_ref,
                     lb_ref)
    nrm = jnp.sqrt(jnp.sum(hn * hn, axis=1, keepdims=True))
    o_ref[...] = hn / jnp.maximum(nrm, 1e-12)


def _tc_layer(h2, agg2, inv2d, ws, wn, b, g, lb, final):
    in_specs = [
        pl.BlockSpec((2, _RB, _HH), lambda i: (0, i, 0)),
        pl.BlockSpec((2, _RB, _HH), lambda i: (0, i, 0)),
        pl.BlockSpec((_RB, 1), lambda i: (i, 0)),
        pl.BlockSpec((_H, _H), lambda i: (0, 0)),
        pl.BlockSpec((_H, _H), lambda i: (0, 0)),
        pl.BlockSpec((1, _H), lambda i: (0, 0)),
        pl.BlockSpec((1, _H), lambda i: (0, 0)),
        pl.BlockSpec((1, _H), lambda i: (0, 0)),
    ]
    if final:
        out_specs = pl.BlockSpec((_RB, _H), lambda i: (i, 0))
        out_shape = jax.ShapeDtypeStruct((_N, _H), jnp.float32)
        body = _final_body
    else:
        out_specs = pl.BlockSpec((2, _RB, _HH), lambda i: (0, i, 0))
        out_shape = jax.ShapeDtypeStruct((2, _N, _HH), jnp.float32)
        body = _layer_body
    return pl.pallas_call(
        body, grid=(10,), in_specs=in_specs,
        out_specs=out_specs, out_shape=out_shape,
    )(h2, agg2, inv2d, ws, wn, b, g, lb)


def _mlp_body(hi_ref, hj_ref, w1_ref, b1_ref, w2_ref, b2_ref, w3_ref, b3_ref,
              o_ref):
    hi = hi_ref[...]
    hj = hj_ref[...]
    feat = jnp.concatenate([jnp.abs(hi - hj), hi * hj], axis=1)
    z = jnp.maximum(
        jnp.dot(feat, w1_ref[...], preferred_element_type=jnp.float32)
        + b1_ref[...], 0.0)
    z = jnp.maximum(
        jnp.dot(z, w2_ref[...], preferred_element_type=jnp.float32)
        + b2_ref[...], 0.0)
    o_ref[...] = (jnp.sum(z * w3_ref[...], axis=1, keepdims=True)
                  + b3_ref[...])


def _tc_mlp(HI, HJ, W1, b1, W2, b2, w3row, b3):
    return pl.pallas_call(
        _mlp_body,
        grid=(_PPAD // _MB,),
        in_specs=[
            pl.BlockSpec((_MB, _H), lambda i: (i, 0)),
            pl.BlockSpec((_MB, _H), lambda i: (i, 0)),
            pl.BlockSpec((2 * _H, _HID), lambda i: (0, 0)),
            pl.BlockSpec((1, _HID), lambda i: (0, 0)),
            pl.BlockSpec((_HID, _HID), lambda i: (0, 0)),
            pl.BlockSpec((1, _HID), lambda i: (0, 0)),
            pl.BlockSpec((1, _HID), lambda i: (0, 0)),
            pl.BlockSpec((1, 1), lambda i: (0, 0)),
        ],
        out_specs=pl.BlockSpec((_MB, 1), lambda i: (i, 0)),
        out_shape=jax.ShapeDtypeStruct((_PPAD, 1), jnp.float32),
    )(HI, HJ, W1, b1, W2, b2, w3row, b3)


def kernel(X, edge_index, i_idx, j_idx, W_inp, b_inp, Ws_self, bs_self,
           Ws_nei, bs_nei, ln_g, ln_b, W1, b1, W2, b2, W3, b3):
    src = edge_index[0]
    dst = edge_index[1]
    epad = _EPAD - _E
    src_slab = jnp.concatenate(
        [src, jnp.zeros((epad,), jnp.int32)]).reshape(_NS * _ECN, _ECH)
    dst_slab = jnp.concatenate(
        [dst, jnp.full((epad,), _N, jnp.int32)]).reshape(_NS * _ECN, _ECH)
    ppad = _PPAD - _P
    i_slab = jnp.concatenate(
        [i_idx, jnp.zeros((ppad,), jnp.int32)]).reshape(_NW * _PCN, _PCH)
    j_slab = jnp.concatenate(
        [j_idx, jnp.zeros((ppad,), jnp.int32)]).reshape(_NW * _PCN, _PCH)

    h2 = _tc_input(X, W_inp, b_inp.reshape(1, _H))
    invdeg = _deg_kernel(dst_slab)
    inv2d = invdeg[:_N].reshape(_N, 1)

    for l in range(_L):
        agg_flat = _agg_kernel(h2.reshape(2 * _N, _HH), src_slab, dst_slab)
        agg2 = agg_flat.reshape(2, _APAD, _HH)
        bsum = (bs_self[l] + bs_nei[l]).reshape(1, _H)
        if l < _L - 1:
            h2 = _tc_layer(h2, agg2, inv2d, Ws_self[l], Ws_nei[l], bsum,
                           ln_g[l].reshape(1, _H), ln_b[l].reshape(1, _H),
                           final=False)
        else:
            Hn = _tc_layer(h2, agg2, inv2d, Ws_self[l], Ws_nei[l], bsum,
                           ln_g[l].reshape(1, _H), ln_b[l].reshape(1, _H),
                           final=True)

    HI, HJ = _pair_kernel(Hn, i_slab, j_slab)
    out2d = _tc_mlp(HI, HJ, W1, b1.reshape(1, _HID), W2, b2.reshape(1, _HID),
                    W3.reshape(1, _HID), b3.reshape(1, 1))
    logits = out2d[:_P, 0]
    return (Hn, logits)


# speedup vs baseline: 4.5475x; 4.4807x over previous
import functools

import jax
import jax.numpy as jnp
from jax import lax
from jax.experimental import pallas as pl
from jax.experimental.pallas import tpu as pltpu
from jax.experimental.pallas import tpu_sc as plsc

_N = 10000
_E = 320000
_D = 128
_H = 256
_HH = 128
_L = 3
_P = 100000
_HID = 128

_NC = 2
_NS = 16
_NW = _NC * _NS

_ECH = 128
_ECN = 160
_SCL = 32
_SCN = _ECN // _SCL
_EPT = _ECH * _ECN
_EPAD = _NS * _EPT

_APAD = 10240
_RPT = _APAD // _NS

_PCH = 128
_PCN = 32
_PPT = _PCH * _PCN
_PPAD = _NW * _PPT

_mesh = plsc.VectorSubcoreMesh(
    core_axis_name="c", subcore_axis_name="s",
    num_cores=_NC, num_subcores=_NS)


@functools.partial(
    pl.kernel,
    out_type=jax.ShapeDtypeStruct((_APAD,), jnp.float32),
    mesh=_mesh,
    scratch_types=[
        pltpu.VMEM((_SCL, _ECH), jnp.int32),
        pltpu.VMEM((_ECH,), jnp.float32),
        pltpu.VMEM((_RPT,), jnp.float32),
        pltpu.VMEM_SHARED((_APAD,), jnp.float32),
    ],
)
def _deg_kernel(dst_hbm, invdeg_hbm, dst_v, ones_v, work_v, dacc):
    c = lax.axis_index("c")
    t = lax.axis_index("s")

    @pl.when(c == 0)
    def _():
        def fz(i, carry):
            work_v[pl.ds(i * 16, 16)] = jnp.zeros((16,), jnp.float32)
            return carry
        lax.fori_loop(0, _RPT // 16, fz, 0)

        def fo(i, carry):
            ones_v[pl.ds(i * 16, 16)] = jnp.full((16,), 1.0, jnp.float32)
            return carry
        lax.fori_loop(0, _ECH // 16, fo, 0)

        pltpu.sync_copy(work_v, dacc.at[pl.ds(t * _RPT, _RPT)])
        plsc.subcore_barrier()

        def superchunk(s, carry):
            pltpu.sync_copy(
                dst_hbm.at[pl.ds(t * _ECN + s * _SCL, _SCL)], dst_v)

            def body(j, carry2):
                pltpu.sync_copy(ones_v, dacc.at[dst_v.at[j]], add=True)
                return carry2
            lax.fori_loop(0, _SCL, body, 0)
            return carry
        lax.fori_loop(0, _SCN, superchunk, 0)
        plsc.subcore_barrier()

        pltpu.sync_copy(dacc.at[pl.ds(t * _RPT, _RPT)], work_v)

        def finv(i, carry):
            v = work_v[pl.ds(i * 16, 16)]
            work_v[pl.ds(i * 16, 16)] = 1.0 / jnp.maximum(v, 1.0)
            return carry
        lax.fori_loop(0, _RPT // 16, finv, 0)
        pltpu.sync_copy(work_v, invdeg_hbm.at[pl.ds(t * _RPT, _RPT)])


@functools.partial(
    pl.kernel,
    out_type=jax.ShapeDtypeStruct((2 * _APAD, _HH), jnp.float32),
    mesh=_mesh,
    scratch_types=[
        pltpu.VMEM((_SCL, _ECH), jnp.int32),
        pltpu.VMEM((_SCL, _ECH), jnp.int32),
        pltpu.VMEM((_ECH, _HH), jnp.float32),
        pltpu.VMEM((_ECH, _HH), jnp.float32),
        pltpu.VMEM_SHARED((_APAD, _HH), jnp.float32),
        pltpu.SemaphoreType.DMA,
        pltpu.SemaphoreType.DMA,
    ],
)
def _agg_kernel(h2_hbm, src_hbm, dst_hbm, out_hbm,
                src_v, dst_v, rows0, rows1, acc, sem0, sem1):
    c = lax.axis_index("c")
    t = lax.axis_index("s")

    def fz(i, carry):
        r = i // 8
        col = (i - r * 8) * 16
        rows0[r, pl.ds(col, 16)] = jnp.zeros((16,), jnp.float32)
        return carry
    lax.fori_loop(0, _ECH * 8, fz, 0)
    for k in range(_RPT // _ECH):
        pltpu.sync_copy(rows0, acc.at[pl.ds(t * _RPT + k * _ECH, _ECH)])
    plsc.subcore_barrier()

    off = c * _N
    base = t * _ECN

    def superchunk(s, carry):
        pltpu.sync_copy(src_hbm.at[pl.ds(base + s * _SCL, _SCL)], src_v)
        pltpu.sync_copy(dst_hbm.at[pl.ds(base + s * _SCL, _SCL)], dst_v)

        def fs(i, carry2):
            r = i // 8
            col = (i - r * 8) * 16
            src_v[r, pl.ds(col, 16)] = src_v[r, pl.ds(col, 16)] + off
            return carry2
        lax.fori_loop(0, _SCL * 8, fs, 0)

        def g_start(j, buf, sem):
            pltpu.make_async_copy(h2_hbm.at[src_v.at[j]], buf, sem).start()

        def g_wait(j, buf, sem):
            pltpu.make_async_copy(h2_hbm.at[src_v.at[j]], buf, sem).wait()

        g_start(0, rows0, sem0)
        g_start(1, rows1, sem1)

        def body(k, carry2):
            j0 = 2 * k
            g_wait(j0, rows0, sem0)

            @pl.when(j0 + 2 < _SCL)
            def _():
                g_start(j0 + 2, rows0, sem0)
            pltpu.sync_copy(rows0, acc.at[dst_v.at[j0]], add=True)

            g_wait(j0 + 1, rows1, sem1)

            @pl.when(j0 + 3 < _SCL)
            def _():
                g_start(j0 + 3, rows1, sem1)
            pltpu.sync_copy(rows1, acc.at[dst_v.at[j0 + 1]], add=True)
            return carry2
        lax.fori_loop(0, _SCL // 2, body, 0)
        return carry
    lax.fori_loop(0, _SCN, superchunk, 0)
    plsc.subcore_barrier()

    pltpu.sync_copy(acc.at[pl.ds(t * _RPT, _RPT)],
                    out_hbm.at[pl.ds(c * _APAD + t * _RPT, _RPT)])


@functools.partial(
    pl.kernel,
    out_type=(jax.ShapeDtypeStruct((_PPAD, _H), jnp.float32),
              jax.ShapeDtypeStruct((_PPAD, _H), jnp.float32)),
    mesh=_mesh,
    scratch_types=[
        pltpu.VMEM((_PCN, _PCH), jnp.int32),
        pltpu.VMEM((_PCN, _PCH), jnp.int32),
        pltpu.VMEM((_PCH, _H), jnp.float32),
        pltpu.VMEM((_PCH, _H), jnp.float32),
        pltpu.SemaphoreType.DMA,
        pltpu.SemaphoreType.DMA,
    ],
)
def _pair_kernel(hn_hbm, i_hbm, j_hbm, hi_hbm, hj_hbm,
                 iv, jv, bi, bj, si, sj):
    c = lax.axis_index("c")
    s = lax.axis_index("s")
    w = c * _NS + s
    base = w * _PPT

    pltpu.sync_copy(i_hbm.at[pl.ds(w * _PCN, _PCN)], iv)
    pltpu.sync_copy(j_hbm.at[pl.ds(w * _PCN, _PCN)], jv)

    def gi_start(j):
        pltpu.make_async_copy(hn_hbm.at[iv.at[j]], bi, si).start()

    def gi_wait(j):
        pltpu.make_async_copy(hn_hbm.at[iv.at[j]], bi, si).wait()

    def gj_start(j):
        pltpu.make_async_copy(hn_hbm.at[jv.at[j]], bj, sj).start()

    def gj_wait(j):
        pltpu.make_async_copy(hn_hbm.at[jv.at[j]], bj, sj).wait()

    gi_start(0)
    gj_start(0)

    def body(k, carry):
        gi_wait(k)
        pltpu.sync_copy(bi, hi_hbm.at[pl.ds(base + k * _PCH, _PCH)])

        @pl.when(k + 1 < _PCN)
        def _():
            gi_start(k + 1)
        gj_wait(k)
        pltpu.sync_copy(bj, hj_hbm.at[pl.ds(base + k * _PCH, _PCH)])

        @pl.when(k + 1 < _PCN)
        def _():
            gj_start(k + 1)
        return carry
    lax.fori_loop(0, _PCN, body, 0)


_RB = 1000
_MB = 1024


def _inp_body(x_ref, w_ref, b_ref, o_ref):
    o_ref[0] = jnp.maximum(
        jnp.dot(x_ref[...], w_ref[...], preferred_element_type=jnp.float32)
        + b_ref[...], 0.0)


def _tc_input(X, W, b):
    return pl.pallas_call(
        _inp_body,
        grid=(2, 10),
        in_specs=[
            pl.BlockSpec((_RB, _D), lambda h, i: (i, 0)),
            pl.BlockSpec((_D, _HH), lambda h, i: (0, h)),
            pl.BlockSpec((1, _HH), lambda h, i: (0, h)),
        ],
        out_specs=pl.BlockSpec((1, _RB, _HH), lambda h, i: (h, i, 0)),
        out_shape=jax.ShapeDtypeStruct((2, _N, _HH), jnp.float32),
    )(X, W, b)


def _layer_math(h2_ref, agg_ref, inv_ref, ws_ref, wn_ref, b_ref, g_ref, lb_ref):
    h = jnp.concatenate([h2_ref[0], h2_ref[1]], axis=1)
    inv = inv_ref[...]
    m = jnp.concatenate([agg_ref[0] * inv, agg_ref[1] * inv], axis=1)
    out = (jnp.dot(h, ws_ref[...], preferred_element_type=jnp.float32)
           + jnp.dot(m, wn_ref[...], preferred_element_type=jnp.float32)
           + b_ref[...])
    out = jnp.maximum(out, 0.0)
    mu = jnp.mean(out, axis=1, keepdims=True)
    d = out - mu
    var = jnp.mean(d * d, axis=1, keepdims=True)
    return d * lax.rsqrt(var + 1e-5) * g_ref[...] + lb_ref[...]


def _layer_body(h2_ref, agg_ref, inv_ref, ws_ref, wn_ref, b_ref, g_ref,
                lb_ref, o_ref):
    hn = _layer_math(h2_ref, agg_ref, inv_ref, ws_ref, wn_ref, b_ref, g_ref,
                     lb_ref)
    o_ref[0] = hn[:, :_HH]
    o_ref[1] = hn[:, _HH:]


def _final_body(h2_ref, agg_ref, inv_ref, ws_ref, wn_ref, b_ref, g_ref,
                lb_ref, o_ref):
    hn = _layer_math(h2_ref, agg_ref, inv_ref, ws_ref, wn_ref, b_ref, g_ref,
                     lb_ref)
    nrm = jnp.sqrt(jnp.sum(hn * hn, axis=1, keepdims=True))
    o_ref[...] = hn / jnp.maximum(nrm, 1e-12)


def _tc_layer(h2, agg2, inv2d, ws, wn, b, g, lb, final):
    in_specs = [
        pl.BlockSpec((2, _RB, _HH), lambda i: (0, i, 0)),
        pl.BlockSpec((2, _RB, _HH), lambda i: (0, i, 0)),
        pl.BlockSpec((_RB, 1), lambda i: (i, 0)),
        pl.BlockSpec((_H, _H), lambda i: (0, 0)),
        pl.BlockSpec((_H, _H), lambda i: (0, 0)),
        pl.BlockSpec((1, _H), lambda i: (0, 0)),
        pl.BlockSpec((1, _H), lambda i: (0, 0)),
        pl.BlockSpec((1, _H), lambda i: (0, 0)),
    ]
    if final:
        out_specs = pl.BlockSpec((_RB, _H), lambda i: (i, 0))
        out_shape = jax.ShapeDtypeStruct((_N, _H), jnp.float32)
        body = _final_body
    else:
        out_specs = pl.BlockSpec((2, _RB, _HH), lambda i: (0, i, 0))
        out_shape = jax.ShapeDtypeStruct((2, _N, _HH), jnp.float32)
        body = _layer_body
    return pl.pallas_call(
        body, grid=(10,), in_specs=in_specs,
        out_specs=out_specs, out_shape=out_shape,
    )(h2, agg2, inv2d, ws, wn, b, g, lb)


def _mlp_body(hi_ref, hj_ref, w1_ref, b1_ref, w2_ref, b2_ref, w3_ref, b3_ref,
              o_ref):
    hi = hi_ref[...]
    hj = hj_ref[...]
    feat = jnp.concatenate([jnp.abs(hi - hj), hi * hj], axis=1)
    z = jnp.maximum(
        jnp.dot(feat, w1_ref[...], preferred_element_type=jnp.float32)
        + b1_ref[...], 0.0)
    z = jnp.maximum(
        jnp.dot(z, w2_ref[...], preferred_element_type=jnp.float32)
        + b2_ref[...], 0.0)
    o_ref[...] = (jnp.sum(z * w3_ref[...], axis=1, keepdims=True)
                  + b3_ref[...])


def _tc_mlp(HI, HJ, W1, b1, W2, b2, w3row, b3):
    return pl.pallas_call(
        _mlp_body,
        grid=(_PPAD // _MB,),
        in_specs=[
            pl.BlockSpec((_MB, _H), lambda i: (i, 0)),
            pl.BlockSpec((_MB, _H), lambda i: (i, 0)),
            pl.BlockSpec((2 * _H, _HID), lambda i: (0, 0)),
            pl.BlockSpec((1, _HID), lambda i: (0, 0)),
            pl.BlockSpec((_HID, _HID), lambda i: (0, 0)),
            pl.BlockSpec((1, _HID), lambda i: (0, 0)),
            pl.BlockSpec((1, _HID), lambda i: (0, 0)),
            pl.BlockSpec((1, 1), lambda i: (0, 0)),
        ],
        out_specs=pl.BlockSpec((_MB, 1), lambda i: (i, 0)),
        out_shape=jax.ShapeDtypeStruct((_PPAD, 1), jnp.float32),
    )(HI, HJ, W1, b1, W2, b2, w3row, b3)


def kernel(X, edge_index, i_idx, j_idx, W_inp, b_inp, Ws_self, bs_self,
           Ws_nei, bs_nei, ln_g, ln_b, W1, b1, W2, b2, W3, b3):
    src = edge_index[0]
    dst = edge_index[1]
    epad = _EPAD - _E
    esp = (jnp.arange(epad, dtype=jnp.int32) * 37) % _N
    src_slab = jnp.concatenate([src, esp]).reshape(_NS * _ECN, _ECH)
    dsp = _N + (jnp.arange(epad, dtype=jnp.int32) % (_APAD - _N))
    dst_slab = jnp.concatenate([dst, dsp]).reshape(_NS * _ECN, _ECH)
    ppad = _PPAD - _P
    psp = (jnp.arange(ppad, dtype=jnp.int32) * 37) % _N
    i_slab = jnp.concatenate([i_idx, psp]).reshape(_NW * _PCN, _PCH)
    j_slab = jnp.concatenate([j_idx, psp]).reshape(_NW * _PCN, _PCH)

    h2 = _tc_input(X, W_inp, b_inp.reshape(1, _H))
    invdeg = _deg_kernel(dst_slab)
    inv2d = invdeg[:_N].reshape(_N, 1)

    for l in range(_L):
        agg_flat = _agg_kernel(h2.reshape(2 * _N, _HH), src_slab, dst_slab)
        agg2 = agg_flat.reshape(2, _APAD, _HH)
        bsum = (bs_self[l] + bs_nei[l]).reshape(1, _H)
        if l < _L - 1:
            h2 = _tc_layer(h2, agg2, inv2d, Ws_self[l], Ws_nei[l], bsum,
                           ln_g[l].reshape(1, _H), ln_b[l].reshape(1, _H),
                           final=False)
        else:
            Hn = _tc_layer(h2, agg2, inv2d, Ws_self[l], Ws_nei[l], bsum,
                           ln_g[l].reshape(1, _H), ln_b[l].reshape(1, _H),
                           final=True)

    HI, HJ = _pair_kernel(Hn, i_slab, j_slab)
    out2d = _tc_mlp(HI, HJ, W1, b1.reshape(1, _HID), W2, b2.reshape(1, _HID),
                    W3.reshape(1, _HID), b3.reshape(1, 1))
    logits = out2d[:_P, 0]
    return (Hn, logits)


# speedup vs baseline: 4.6255x; 1.0171x over previous
import functools

import jax
import jax.numpy as jnp
from jax import lax
from jax.experimental import pallas as pl
from jax.experimental.pallas import tpu as pltpu
from jax.experimental.pallas import tpu_sc as plsc

_N = 10000
_E = 320000
_D = 128
_H = 256
_HH = 128
_L = 3
_P = 100000
_HID = 128

_NC = 2
_NS = 16
_NW = _NC * _NS

_ECH = 128
_ECN = 160
_SCL = 32
_SCN = _ECN // _SCL
_EPT = _ECH * _ECN
_EPAD = _NS * _EPT

_APAD = 10240
_RPT = _APAD // _NS

_PCH = 128
_PCN = 16
_PPT = _PCH * _PCN
_PPADH = _NW * _PPT
_PPAD = 2 * _PPADH

_mesh = plsc.VectorSubcoreMesh(
    core_axis_name="c", subcore_axis_name="s",
    num_cores=_NC, num_subcores=_NS)


@functools.partial(
    pl.kernel,
    out_type=jax.ShapeDtypeStruct((_APAD,), jnp.float32),
    mesh=_mesh,
    scratch_types=[
        pltpu.VMEM((_SCL, _ECH), jnp.int32),
        pltpu.VMEM((_ECH,), jnp.float32),
        pltpu.VMEM((_RPT,), jnp.float32),
        pltpu.VMEM_SHARED((_APAD,), jnp.float32),
    ],
)
def _deg_kernel(dst_hbm, invdeg_hbm, dst_v, ones_v, work_v, dacc):
    c = lax.axis_index("c")
    t = lax.axis_index("s")

    @pl.when(c == 0)
    def _():
        def fz(i, carry):
            work_v[pl.ds(i * 16, 16)] = jnp.zeros((16,), jnp.float32)
            return carry
        lax.fori_loop(0, _RPT // 16, fz, 0)

        def fo(i, carry):
            ones_v[pl.ds(i * 16, 16)] = jnp.full((16,), 1.0, jnp.float32)
            return carry
        lax.fori_loop(0, _ECH // 16, fo, 0)

        pltpu.sync_copy(work_v, dacc.at[pl.ds(t * _RPT, _RPT)])
        plsc.subcore_barrier()

        def superchunk(s, carry):
            pltpu.sync_copy(
                dst_hbm.at[pl.ds(t * _ECN + s * _SCL, _SCL)], dst_v)

            def body(j, carry2):
                pltpu.sync_copy(ones_v, dacc.at[dst_v.at[j]], add=True)
                return carry2
            lax.fori_loop(0, _SCL, body, 0)
            return carry
        lax.fori_loop(0, _SCN, superchunk, 0)
        plsc.subcore_barrier()

        pltpu.sync_copy(dacc.at[pl.ds(t * _RPT, _RPT)], work_v)

        def finv(i, carry):
            v = work_v[pl.ds(i * 16, 16)]
            work_v[pl.ds(i * 16, 16)] = 1.0 / jnp.maximum(v, 1.0)
            return carry
        lax.fori_loop(0, _RPT // 16, finv, 0)
        pltpu.sync_copy(work_v, invdeg_hbm.at[pl.ds(t * _RPT, _RPT)])


@functools.partial(
    pl.kernel,
    out_type=jax.ShapeDtypeStruct((2 * _APAD, _HH), jnp.float32),
    mesh=_mesh,
    scratch_types=[
        pltpu.VMEM((_SCL, _ECH), jnp.int32),
        pltpu.VMEM((_SCL, _ECH), jnp.int32),
        pltpu.VMEM((_ECH, _HH), jnp.float32),
        pltpu.VMEM((_ECH, _HH), jnp.float32),
        pltpu.VMEM_SHARED((_APAD, _HH), jnp.float32),
        pltpu.SemaphoreType.DMA,
        pltpu.SemaphoreType.DMA,
    ],
)
def _agg_kernel(h2_hbm, src_hbm, dst_hbm, out_hbm,
                src_v, dst_v, rows0, rows1, acc, sem0, sem1):
    c = lax.axis_index("c")
    t = lax.axis_index("s")

    def fz(i, carry):
        r = i // 8
        col = (i - r * 8) * 16
        rows0[r, pl.ds(col, 16)] = jnp.zeros((16,), jnp.float32)
        return carry
    lax.fori_loop(0, _ECH * 8, fz, 0)
    for k in range(_RPT // _ECH):
        pltpu.sync_copy(rows0, acc.at[pl.ds(t * _RPT + k * _ECH, _ECH)])
    plsc.subcore_barrier()

    off = c * _N
    base = t * _ECN

    def superchunk(s, carry):
        pltpu.sync_copy(src_hbm.at[pl.ds(base + s * _SCL, _SCL)], src_v)
        pltpu.sync_copy(dst_hbm.at[pl.ds(base + s * _SCL, _SCL)], dst_v)

        def fs(i, carry2):
            r = i // 8
            col = (i - r * 8) * 16
            src_v[r, pl.ds(col, 16)] = src_v[r, pl.ds(col, 16)] + off
            return carry2
        lax.fori_loop(0, _SCL * 8, fs, 0)

        def g_start(j, buf, sem):
            pltpu.make_async_copy(h2_hbm.at[src_v.at[j]], buf, sem).start()

        def g_wait(j, buf, sem):
            pltpu.make_async_copy(h2_hbm.at[src_v.at[j]], buf, sem).wait()

        g_start(0, rows0, sem0)
        g_start(1, rows1, sem1)

        def body(k, carry2):
            j0 = 2 * k
            g_wait(j0, rows0, sem0)

            @pl.when(j0 + 2 < _SCL)
            def _():
                g_start(j0 + 2, rows0, sem0)
            pltpu.sync_copy(rows0, acc.at[dst_v.at[j0]], add=True)

            g_wait(j0 + 1, rows1, sem1)

            @pl.when(j0 + 3 < _SCL)
            def _():
                g_start(j0 + 3, rows1, sem1)
            pltpu.sync_copy(rows1, acc.at[dst_v.at[j0 + 1]], add=True)
            return carry2
        lax.fori_loop(0, _SCL // 2, body, 0)
        return carry
    lax.fori_loop(0, _SCN, superchunk, 0)
    plsc.subcore_barrier()

    pltpu.sync_copy(acc.at[pl.ds(t * _RPT, _RPT)],
                    out_hbm.at[pl.ds(c * _APAD + t * _RPT, _RPT)])


@functools.partial(
    pl.kernel,
    out_type=(jax.ShapeDtypeStruct((_PPADH, _H), jnp.float32),
              jax.ShapeDtypeStruct((_PPADH, _H), jnp.float32)),
    mesh=_mesh,
    scratch_types=[
        pltpu.VMEM((_PCN, _PCH), jnp.int32),
        pltpu.VMEM((_PCN, _PCH), jnp.int32),
        pltpu.VMEM((_PCH, _H), jnp.float32),
        pltpu.VMEM((_PCH, _H), jnp.float32),
        pltpu.SemaphoreType.DMA,
        pltpu.SemaphoreType.DMA,
    ],
)
def _pair_kernel(hn_hbm, i_hbm, j_hbm, hi_hbm, hj_hbm,
                 iv, jv, bi, bj, si, sj):
    c = lax.axis_index("c")
    s = lax.axis_index("s")
    w = c * _NS + s
    base = w * _PPT

    pltpu.sync_copy(i_hbm.at[pl.ds(w * _PCN, _PCN)], iv)
    pltpu.sync_copy(j_hbm.at[pl.ds(w * _PCN, _PCN)], jv)

    def gi_start(j):
        pltpu.make_async_copy(hn_hbm.at[iv.at[j]], bi, si).start()

    def gi_wait(j):
        pltpu.make_async_copy(hn_hbm.at[iv.at[j]], bi, si).wait()

    def gj_start(j):
        pltpu.make_async_copy(hn_hbm.at[jv.at[j]], bj, sj).start()

    def gj_wait(j):
        pltpu.make_async_copy(hn_hbm.at[jv.at[j]], bj, sj).wait()

    gi_start(0)
    gj_start(0)

    def body(k, carry):
        gi_wait(k)
        pltpu.sync_copy(bi, hi_hbm.at[pl.ds(base + k * _PCH, _PCH)])

        @pl.when(k + 1 < _PCN)
        def _():
            gi_start(k + 1)
        gj_wait(k)
        pltpu.sync_copy(bj, hj_hbm.at[pl.ds(base + k * _PCH, _PCH)])

        @pl.when(k + 1 < _PCN)
        def _():
            gj_start(k + 1)
        return carry
    lax.fori_loop(0, _PCN, body, 0)


_RB = 1000
_MB = 1024


def _inp_body(x_ref, w_ref, b_ref, o_ref):
    o_ref[0] = jnp.maximum(
        jnp.dot(x_ref[...], w_ref[...], preferred_element_type=jnp.float32)
        + b_ref[...], 0.0)


def _tc_input(X, W, b):
    return pl.pallas_call(
        _inp_body,
        grid=(2, 10),
        in_specs=[
            pl.BlockSpec((_RB, _D), lambda h, i: (i, 0)),
            pl.BlockSpec((_D, _HH), lambda h, i: (0, h)),
            pl.BlockSpec((1, _HH), lambda h, i: (0, h)),
        ],
        out_specs=pl.BlockSpec((1, _RB, _HH), lambda h, i: (h, i, 0)),
        out_shape=jax.ShapeDtypeStruct((2, _N, _HH), jnp.float32),
    )(X, W, b)


def _layer_math(h2_ref, agg_ref, inv_ref, ws_ref, wn_ref, b_ref, g_ref, lb_ref):
    h = jnp.concatenate([h2_ref[0], h2_ref[1]], axis=1)
    inv = inv_ref[...]
    m = jnp.concatenate([agg_ref[0] * inv, agg_ref[1] * inv], axis=1)
    out = (jnp.dot(h, ws_ref[...], preferred_element_type=jnp.float32)
           + jnp.dot(m, wn_ref[...], preferred_element_type=jnp.float32)
           + b_ref[...])
    out = jnp.maximum(out, 0.0)
    mu = jnp.mean(out, axis=1, keepdims=True)
    d = out - mu
    var = jnp.mean(d * d, axis=1, keepdims=True)
    return d * lax.rsqrt(var + 1e-5) * g_ref[...] + lb_ref[...]


def _layer_body(h2_ref, agg_ref, inv_ref, ws_ref, wn_ref, b_ref, g_ref,
                lb_ref, o_ref):
    hn = _layer_math(h2_ref, agg_ref, inv_ref, ws_ref, wn_ref, b_ref, g_ref,
                     lb_ref)
    o_ref[0] = hn[:, :_HH]
    o_ref[1] = hn[:, _HH:]


def _final_body(h2_ref, agg_ref, inv_ref, ws_ref, wn_ref, b_ref, g_ref,
                lb_ref, o_ref):
    hn = _layer_math(h2_ref, agg_ref, inv_ref, ws_ref, wn_ref, b_ref, g_ref,
                     lb_ref)
    nrm = jnp.sqrt(jnp.sum(hn * hn, axis=1, keepdims=True))
    o_ref[...] = hn / jnp.maximum(nrm, 1e-12)


def _tc_layer(h2, agg2, inv2d, ws, wn, b, g, lb, final):
    in_specs = [
        pl.BlockSpec((2, _RB, _HH), lambda i: (0, i, 0)),
        pl.BlockSpec((2, _RB, _HH), lambda i: (0, i, 0)),
        pl.BlockSpec((_RB, 1), lambda i: (i, 0)),
        pl.BlockSpec((_H, _H), lambda i: (0, 0)),
        pl.BlockSpec((_H, _H), lambda i: (0, 0)),
        pl.BlockSpec((1, _H), lambda i: (0, 0)),
        pl.BlockSpec((1, _H), lambda i: (0, 0)),
        pl.BlockSpec((1, _H), lambda i: (0, 0)),
    ]
    if final:
        out_specs = pl.BlockSpec((_RB, _H), lambda i: (i, 0))
        out_shape = jax.ShapeDtypeStruct((_N, _H), jnp.float32)
        body = _final_body
    else:
        out_specs = pl.BlockSpec((2, _RB, _HH), lambda i: (0, i, 0))
        out_shape = jax.ShapeDtypeStruct((2, _N, _HH), jnp.float32)
        body = _layer_body
    return pl.pallas_call(
        body, grid=(10,), in_specs=in_specs,
        out_specs=out_specs, out_shape=out_shape,
    )(h2, agg2, inv2d, ws, wn, b, g, lb)


def _mlp_body(hi_ref, hj_ref, w1_ref, b1_ref, w2_ref, b2_ref, w3_ref, b3_ref,
              o_ref):
    hi = hi_ref[...]
    hj = hj_ref[...]
    feat = jnp.concatenate([jnp.abs(hi - hj), hi * hj], axis=1)
    z = jnp.maximum(
        jnp.dot(feat, w1_ref[...], preferred_element_type=jnp.float32)
        + b1_ref[...], 0.0)
    z = jnp.maximum(
        jnp.dot(z, w2_ref[...], preferred_element_type=jnp.float32)
        + b2_ref[...], 0.0)
    o_ref[...] = (jnp.sum(z * w3_ref[...], axis=1, keepdims=True)
                  + b3_ref[...])


def _tc_mlp(HI, HJ, W1, b1, W2, b2, w3row, b3):
    return pl.pallas_call(
        _mlp_body,
        grid=(_PPADH // _MB,),
        in_specs=[
            pl.BlockSpec((_MB, _H), lambda i: (i, 0)),
            pl.BlockSpec((_MB, _H), lambda i: (i, 0)),
            pl.BlockSpec((2 * _H, _HID), lambda i: (0, 0)),
            pl.BlockSpec((1, _HID), lambda i: (0, 0)),
            pl.BlockSpec((_HID, _HID), lambda i: (0, 0)),
            pl.BlockSpec((1, _HID), lambda i: (0, 0)),
            pl.BlockSpec((1, _HID), lambda i: (0, 0)),
            pl.BlockSpec((1, 1), lambda i: (0, 0)),
        ],
        out_specs=pl.BlockSpec((_MB, 1), lambda i: (i, 0)),
        out_shape=jax.ShapeDtypeStruct((_PPADH, 1), jnp.float32),
    )(HI, HJ, W1, b1, W2, b2, w3row, b3)


def kernel(X, edge_index, i_idx, j_idx, W_inp, b_inp, Ws_self, bs_self,
           Ws_nei, bs_nei, ln_g, ln_b, W1, b1, W2, b2, W3, b3):
    src = edge_index[0]
    dst = edge_index[1]
    epad = _EPAD - _E
    esp = (jnp.arange(epad, dtype=jnp.int32) * 37) % _N
    src_slab = jnp.concatenate([src, esp]).reshape(_NS * _ECN, _ECH)
    dsp = _N + (jnp.arange(epad, dtype=jnp.int32) % (_APAD - _N))
    dst_slab = jnp.concatenate([dst, dsp]).reshape(_NS * _ECN, _ECH)
    ppad = _PPAD - _P
    psp = (jnp.arange(ppad, dtype=jnp.int32) * 37) % _N
    i_slab = jnp.concatenate([i_idx, psp]).reshape(2, _NW * _PCN, _PCH)
    j_slab = jnp.concatenate([j_idx, psp]).reshape(2, _NW * _PCN, _PCH)

    h2 = _tc_input(X, W_inp, b_inp.reshape(1, _H))
    invdeg = _deg_kernel(dst_slab)
    inv2d = invdeg[:_N].reshape(_N, 1)

    for l in range(_L):
        agg_flat = _agg_kernel(h2.reshape(2 * _N, _HH), src_slab, dst_slab)
        agg2 = agg_flat.reshape(2, _APAD, _HH)
        bsum = (bs_self[l] + bs_nei[l]).reshape(1, _H)
        if l < _L - 1:
            h2 = _tc_layer(h2, agg2, inv2d, Ws_self[l], Ws_nei[l], bsum,
                           ln_g[l].reshape(1, _H), ln_b[l].reshape(1, _H),
                           final=False)
        else:
            Hn = _tc_layer(h2, agg2, inv2d, Ws_self[l], Ws_nei[l], bsum,
                           ln_g[l].reshape(1, _H), ln_b[l].reshape(1, _H),
                           final=True)

    b1r = b1.reshape(1, _HID)
    b2r = b2.reshape(1, _HID)
    w3r = W3.reshape(1, _HID)
    b3r = b3.reshape(1, 1)
    HI1, HJ1 = _pair_kernel(Hn, i_slab[0], j_slab[0])
    HI2, HJ2 = _pair_kernel(Hn, i_slab[1], j_slab[1])
    o1 = _tc_mlp(HI1, HJ1, W1, b1r, W2, b2r, w3r, b3r)
    o2 = _tc_mlp(HI2, HJ2, W1, b1r, W2, b2r, w3r, b3r)
    logits = jnp.concatenate([o1[:, 0], o2[:, 0]])[:_P]
    return (Hn, logits)


# speedup vs baseline: 4.6333x; 1.0017x over previous
import functools

import jax
import jax.numpy as jnp
from jax import lax
from jax.experimental import pallas as pl
from jax.experimental.pallas import tpu as pltpu
from jax.experimental.pallas import tpu_sc as plsc

_N = 10000
_E = 320000
_D = 128
_H = 256
_HH = 128
_L = 3
_P = 100000
_HID = 128

_NC = 2
_NS = 16
_NW = _NC * _NS

_ECH = 128
_ECN = 160
_SCL = 32
_SCN = _ECN // _SCL
_EPT = _ECH * _ECN
_EPAD = _NS * _EPT

_APAD = 10240
_RPT = _APAD // _NS

_PCH = 128
_PCN = 8
_PPT = _PCH * _PCN
_PPADH = _NW * _PPT
_PSL = 4
_PPAD = _PSL * _PPADH

_mesh = plsc.VectorSubcoreMesh(
    core_axis_name="c", subcore_axis_name="s",
    num_cores=_NC, num_subcores=_NS)


@functools.partial(
    pl.kernel,
    out_type=jax.ShapeDtypeStruct((_APAD,), jnp.float32),
    mesh=_mesh,
    scratch_types=[
        pltpu.VMEM((_SCL, _ECH), jnp.int32),
        pltpu.VMEM((_ECH,), jnp.float32),
        pltpu.VMEM((_RPT,), jnp.float32),
        pltpu.VMEM_SHARED((_APAD,), jnp.float32),
    ],
)
def _deg_kernel(dst_hbm, invdeg_hbm, dst_v, ones_v, work_v, dacc):
    c = lax.axis_index("c")
    t = lax.axis_index("s")

    @pl.when(c == 0)
    def _():
        def fz(i, carry):
            work_v[pl.ds(i * 16, 16)] = jnp.zeros((16,), jnp.float32)
            return carry
        lax.fori_loop(0, _RPT // 16, fz, 0)

        def fo(i, carry):
            ones_v[pl.ds(i * 16, 16)] = jnp.full((16,), 1.0, jnp.float32)
            return carry
        lax.fori_loop(0, _ECH // 16, fo, 0)

        pltpu.sync_copy(work_v, dacc.at[pl.ds(t * _RPT, _RPT)])
        plsc.subcore_barrier()

        def superchunk(s, carry):
            pltpu.sync_copy(
                dst_hbm.at[pl.ds(t * _ECN + s * _SCL, _SCL)], dst_v)

            def body(j, carry2):
                pltpu.sync_copy(ones_v, dacc.at[dst_v.at[j]], add=True)
                return carry2
            lax.fori_loop(0, _SCL, body, 0)
            return carry
        lax.fori_loop(0, _SCN, superchunk, 0)
        plsc.subcore_barrier()

        pltpu.sync_copy(dacc.at[pl.ds(t * _RPT, _RPT)], work_v)

        def finv(i, carry):
            v = work_v[pl.ds(i * 16, 16)]
            work_v[pl.ds(i * 16, 16)] = 1.0 / jnp.maximum(v, 1.0)
            return carry
        lax.fori_loop(0, _RPT // 16, finv, 0)
        pltpu.sync_copy(work_v, invdeg_hbm.at[pl.ds(t * _RPT, _RPT)])


@functools.partial(
    pl.kernel,
    out_type=jax.ShapeDtypeStruct((2 * _APAD, _HH), jnp.float32),
    mesh=_mesh,
    scratch_types=[
        pltpu.VMEM((_SCL, _ECH), jnp.int32),
        pltpu.VMEM((_SCL, _ECH), jnp.int32),
        pltpu.VMEM((_ECH, _HH), jnp.float32),
        pltpu.VMEM((_ECH, _HH), jnp.float32),
        pltpu.VMEM_SHARED((_APAD, _HH), jnp.float32),
        pltpu.SemaphoreType.DMA,
        pltpu.SemaphoreType.DMA,
    ],
)
def _agg_kernel(h2_hbm, src_hbm, dst_hbm, out_hbm,
                src_v, dst_v, rows0, rows1, acc, sem0, sem1):
    c = lax.axis_index("c")
    t = lax.axis_index("s")

    def fz(i, carry):
        r = i // 8
        col = (i - r * 8) * 16
        rows0[r, pl.ds(col, 16)] = jnp.zeros((16,), jnp.float32)
        return carry
    lax.fori_loop(0, _ECH * 8, fz, 0)
    for k in range(_RPT // _ECH):
        pltpu.sync_copy(rows0, acc.at[pl.ds(t * _RPT + k * _ECH, _ECH)])
    plsc.subcore_barrier()

    off = c * _N
    base = t * _ECN

    def superchunk(s, carry):
        pltpu.sync_copy(src_hbm.at[pl.ds(base + s * _SCL, _SCL)], src_v)
        pltpu.sync_copy(dst_hbm.at[pl.ds(base + s * _SCL, _SCL)], dst_v)

        def fs(i, carry2):
            r = i // 8
            col = (i - r * 8) * 16
            src_v[r, pl.ds(col, 16)] = src_v[r, pl.ds(col, 16)] + off
            return carry2
        lax.fori_loop(0, _SCL * 8, fs, 0)

        def g_start(j, buf, sem):
            pltpu.make_async_copy(h2_hbm.at[src_v.at[j]], buf, sem).start()

        def g_wait(j, buf, sem):
            pltpu.make_async_copy(h2_hbm.at[src_v.at[j]], buf, sem).wait()

        g_start(0, rows0, sem0)
        g_start(1, rows1, sem1)

        def body(k, carry2):
            j0 = 2 * k
            g_wait(j0, rows0, sem0)

            @pl.when(j0 + 2 < _SCL)
            def _():
                g_start(j0 + 2, rows0, sem0)
            pltpu.sync_copy(rows0, acc.at[dst_v.at[j0]], add=True)

            g_wait(j0 + 1, rows1, sem1)

            @pl.when(j0 + 3 < _SCL)
            def _():
                g_start(j0 + 3, rows1, sem1)
            pltpu.sync_copy(rows1, acc.at[dst_v.at[j0 + 1]], add=True)
            return carry2
        lax.fori_loop(0, _SCL // 2, body, 0)
        return carry
    lax.fori_loop(0, _SCN, superchunk, 0)
    plsc.subcore_barrier()

    pltpu.sync_copy(acc.at[pl.ds(t * _RPT, _RPT)],
                    out_hbm.at[pl.ds(c * _APAD + t * _RPT, _RPT)])


@functools.partial(
    pl.kernel,
    out_type=(jax.ShapeDtypeStruct((_PPADH, _H), jnp.float32),
              jax.ShapeDtypeStruct((_PPADH, _H), jnp.float32)),
    mesh=_mesh,
    scratch_types=[
        pltpu.VMEM((_PCN, _PCH), jnp.int32),
        pltpu.VMEM((_PCN, _PCH), jnp.int32),
        pltpu.VMEM((_PCH, _H), jnp.float32),
        pltpu.VMEM((_PCH, _H), jnp.float32),
        pltpu.SemaphoreType.DMA,
        pltpu.SemaphoreType.DMA,
    ],
)
def _pair_kernel(hn_hbm, i_hbm, j_hbm, hi_hbm, hj_hbm,
                 iv, jv, bi, bj, si, sj):
    c = lax.axis_index("c")
    s = lax.axis_index("s")
    w = c * _NS + s
    base = w * _PPT

    pltpu.sync_copy(i_hbm.at[pl.ds(w * _PCN, _PCN)], iv)
    pltpu.sync_copy(j_hbm.at[pl.ds(w * _PCN, _PCN)], jv)

    def gi_start(j):
        pltpu.make_async_copy(hn_hbm.at[iv.at[j]], bi, si).start()

    def gi_wait(j):
        pltpu.make_async_copy(hn_hbm.at[iv.at[j]], bi, si).wait()

    def gj_start(j):
        pltpu.make_async_copy(hn_hbm.at[jv.at[j]], bj, sj).start()

    def gj_wait(j):
        pltpu.make_async_copy(hn_hbm.at[jv.at[j]], bj, sj).wait()

    gi_start(0)
    gj_start(0)

    def body(k, carry):
        gi_wait(k)
        pltpu.sync_copy(bi, hi_hbm.at[pl.ds(base + k * _PCH, _PCH)])

        @pl.when(k + 1 < _PCN)
        def _():
            gi_start(k + 1)
        gj_wait(k)
        pltpu.sync_copy(bj, hj_hbm.at[pl.ds(base + k * _PCH, _PCH)])

        @pl.when(k + 1 < _PCN)
        def _():
            gj_start(k + 1)
        return carry
    lax.fori_loop(0, _PCN, body, 0)


_RB = 1000
_MB = 1024


def _inp_body(x_ref, w_ref, b_ref, o_ref):
    o_ref[0] = jnp.maximum(
        jnp.dot(x_ref[...], w_ref[...], preferred_element_type=jnp.float32)
        + b_ref[...], 0.0)


def _tc_input(X, W, b):
    return pl.pallas_call(
        _inp_body,
        grid=(2, 10),
        in_specs=[
            pl.BlockSpec((_RB, _D), lambda h, i: (i, 0)),
            pl.BlockSpec((_D, _HH), lambda h, i: (0, h)),
            pl.BlockSpec((1, _HH), lambda h, i: (0, h)),
        ],
        out_specs=pl.BlockSpec((1, _RB, _HH), lambda h, i: (h, i, 0)),
        out_shape=jax.ShapeDtypeStruct((2, _N, _HH), jnp.float32),
    )(X, W, b)


def _layer_math(h2_ref, agg_ref, inv_ref, ws_ref, wn_ref, b_ref, g_ref, lb_ref):
    h = jnp.concatenate([h2_ref[0], h2_ref[1]], axis=1)
    inv = inv_ref[...]
    m = jnp.concatenate([agg_ref[0] * inv, agg_ref[1] * inv], axis=1)
    out = (jnp.dot(h, ws_ref[...], preferred_element_type=jnp.float32)
           + jnp.dot(m, wn_ref[...], preferred_element_type=jnp.float32)
           + b_ref[...])
    out = jnp.maximum(out, 0.0)
    mu = jnp.mean(out, axis=1, keepdims=True)
    d = out - mu
    var = jnp.mean(d * d, axis=1, keepdims=True)
    return d * lax.rsqrt(var + 1e-5) * g_ref[...] + lb_ref[...]


def _layer_body(h2_ref, agg_ref, inv_ref, ws_ref, wn_ref, b_ref, g_ref,
                lb_ref, o_ref):
    hn = _layer_math(h2_ref, agg_ref, inv_ref, ws_ref, wn_ref, b_ref, g_ref,
                     lb_ref)
    o_ref[0] = hn[:, :_HH]
    o_ref[1] = hn[:, _HH:]


def _final_body(h2_ref, agg_ref, inv_ref, ws_ref, wn_ref, b_ref, g_ref,
                lb_ref, o_ref):
    hn = _layer_math(h2_ref, agg_ref, inv_ref, ws_ref, wn_ref, b_ref, g_ref,
                     lb_ref)
    nrm = jnp.sqrt(jnp.sum(hn * hn, axis=1, keepdims=True))
    o_ref[...] = hn / jnp.maximum(nrm, 1e-12)


def _tc_layer(h2, agg2, inv2d, ws, wn, b, g, lb, final):
    in_specs = [
        pl.BlockSpec((2, _RB, _HH), lambda i: (0, i, 0)),
        pl.BlockSpec((2, _RB, _HH), lambda i: (0, i, 0)),
        pl.BlockSpec((_RB, 1), lambda i: (i, 0)),
        pl.BlockSpec((_H, _H), lambda i: (0, 0)),
        pl.BlockSpec((_H, _H), lambda i: (0, 0)),
        pl.BlockSpec((1, _H), lambda i: (0, 0)),
        pl.BlockSpec((1, _H), lambda i: (0, 0)),
        pl.BlockSpec((1, _H), lambda i: (0, 0)),
    ]
    if final:
        out_specs = pl.BlockSpec((_RB, _H), lambda i: (i, 0))
        out_shape = jax.ShapeDtypeStruct((_N, _H), jnp.float32)
        body = _final_body
    else:
        out_specs = pl.BlockSpec((2, _RB, _HH), lambda i: (0, i, 0))
        out_shape = jax.ShapeDtypeStruct((2, _N, _HH), jnp.float32)
        body = _layer_body
    return pl.pallas_call(
        body, grid=(10,), in_specs=in_specs,
        out_specs=out_specs, out_shape=out_shape,
    )(h2, agg2, inv2d, ws, wn, b, g, lb)


def _mlp_body(hi_ref, hj_ref, w1_ref, b1_ref, w2_ref, b2_ref, w3_ref, b3_ref,
              o_ref):
    hi = hi_ref[...]
    hj = hj_ref[...]
    feat = jnp.concatenate([jnp.abs(hi - hj), hi * hj], axis=1)
    z = jnp.maximum(
        jnp.dot(feat, w1_ref[...], preferred_element_type=jnp.float32)
        + b1_ref[...], 0.0)
    z = jnp.maximum(
        jnp.dot(z, w2_ref[...], preferred_element_type=jnp.float32)
        + b2_ref[...], 0.0)
    o_ref[...] = (jnp.sum(z * w3_ref[...], axis=1, keepdims=True)
                  + b3_ref[...])


def _tc_mlp(HI, HJ, W1, b1, W2, b2, w3row, b3):
    return pl.pallas_call(
        _mlp_body,
        grid=(_PPADH // _MB,),
        in_specs=[
            pl.BlockSpec((_MB, _H), lambda i: (i, 0)),
            pl.BlockSpec((_MB, _H), lambda i: (i, 0)),
            pl.BlockSpec((2 * _H, _HID), lambda i: (0, 0)),
            pl.BlockSpec((1, _HID), lambda i: (0, 0)),
            pl.BlockSpec((_HID, _HID), lambda i: (0, 0)),
            pl.BlockSpec((1, _HID), lambda i: (0, 0)),
            pl.BlockSpec((1, _HID), lambda i: (0, 0)),
            pl.BlockSpec((1, 1), lambda i: (0, 0)),
        ],
        out_specs=pl.BlockSpec((_MB, 1), lambda i: (i, 0)),
        out_shape=jax.ShapeDtypeStruct((_PPADH, 1), jnp.float32),
    )(HI, HJ, W1, b1, W2, b2, w3row, b3)


def kernel(X, edge_index, i_idx, j_idx, W_inp, b_inp, Ws_self, bs_self,
           Ws_nei, bs_nei, ln_g, ln_b, W1, b1, W2, b2, W3, b3):
    src = edge_index[0]
    dst = edge_index[1]
    epad = _EPAD - _E
    esp = (jnp.arange(epad, dtype=jnp.int32) * 37) % _N
    src_slab = jnp.concatenate([src, esp]).reshape(_NS * _ECN, _ECH)
    dsp = _N + (jnp.arange(epad, dtype=jnp.int32) % (_APAD - _N))
    dst_slab = jnp.concatenate([dst, dsp]).reshape(_NS * _ECN, _ECH)
    ppad = _PPAD - _P
    psp = (jnp.arange(ppad, dtype=jnp.int32) * 37) % _N
    i_slab = jnp.concatenate([i_idx, psp]).reshape(_PSL, _NW * _PCN, _PCH)
    j_slab = jnp.concatenate([j_idx, psp]).reshape(_PSL, _NW * _PCN, _PCH)

    h2 = _tc_input(X, W_inp, b_inp.reshape(1, _H))
    invdeg = _deg_kernel(dst_slab)
    inv2d = invdeg[:_N].reshape(_N, 1)

    for l in range(_L):
        agg_flat = _agg_kernel(h2.reshape(2 * _N, _HH), src_slab, dst_slab)
        agg2 = agg_flat.reshape(2, _APAD, _HH)
        bsum = (bs_self[l] + bs_nei[l]).reshape(1, _H)
        if l < _L - 1:
            h2 = _tc_layer(h2, agg2, inv2d, Ws_self[l], Ws_nei[l], bsum,
                           ln_g[l].reshape(1, _H), ln_b[l].reshape(1, _H),
                           final=False)
        else:
            Hn = _tc_layer(h2, agg2, inv2d, Ws_self[l], Ws_nei[l], bsum,
                           ln_g[l].reshape(1, _H), ln_b[l].reshape(1, _H),
                           final=True)

    b1r = b1.reshape(1, _HID)
    b2r = b2.reshape(1, _HID)
    w3r = W3.reshape(1, _HID)
    b3r = b3.reshape(1, 1)
    outs = []
    for q in range(_PSL):
        HIq, HJq = _pair_kernel(Hn, i_slab[q], j_slab[q])
        outs.append(_tc_mlp(HIq, HJq, W1, b1r, W2, b2r, w3r, b3r))
    logits = jnp.concatenate(outs)[:_P, 0]
    return (Hn, logits)
